# baseline (speedup 1.0000x reference)
import jax
import jax.numpy as jnp
from jax.experimental import pallas as pl

_QB = 128
_L = 128
_K = 16
_KN = 8
_BIG = 3.0e38
_IIBIG = 2**30


def _knn_loss_kernel(q_ref, qs_ref, c_ref, s_ref, pos_ref, neg_ref, sm_ref):
    i = pl.program_id(0)
    g = c_ref.shape[1]
    scores_mat = s_ref[...]

    cx = c_ref[0][None]
    cy = c_ref[1][None]
    cz = c_ref[2][None]
    cn = cx * cx + cy * cy + cz * cz

    def make_chain(lo, hi):
        qb = hi - lo
        qx = q_ref[lo:hi, 0:1][:, :, None] * -2.0
        qy = q_ref[lo:hi, 1:2][:, :, None] * -2.0
        qz = q_ref[lo:hi, 2:3][:, :, None] * -2.0
        d2 = cn + (qx * cx + qy * cy + qz * cz)
        gmin0 = jnp.min(d2, axis=2)
        iota_g = jax.lax.broadcasted_iota(jnp.int32, (qb, g), 1)
        iota_l = jax.lax.broadcasted_iota(jnp.int32, (qb, _L), 1)
        qs = qs_ref[lo:hi, 0:1]
        z = jnp.zeros_like(qs)
        init = (gmin0, jnp.full_like(qs, -_BIG),
                jnp.full_like(qs, -1, dtype=jnp.int32), z, z, z)

        def step(t, carry):
            gmin, kd, ki, sum_pos, sum_neg, sum8 = carry
            m = jnp.min(gmin, axis=1, keepdims=True)
            gsel_i = jnp.min(jnp.where(gmin == m, iota_g, _IIBIG),
                             axis=1, keepdims=True)
            selg = iota_g == gsel_i
            penal = (1.0 - selg.astype(jnp.float32)) * _BIG
            rowd = jnp.min(d2 + penal[:, :, None], axis=1)
            gidx = gsel_i * _L + iota_l
            valid = (rowd > kd) | ((rowd == kd) & (gidx > ki))
            l_i = jnp.min(jnp.where(valid & (rowd == m), gidx, _IIBIG),
                          axis=1, keepdims=True)
            srow = jnp.dot(selg.astype(jnp.float32), scores_mat,
                           preferred_element_type=jnp.float32)
            s = jnp.sum(jnp.where(gidx == l_i, srow, 0.0),
                        axis=1, keepdims=True)
            valid_new = (rowd > m) | ((rowd == m) & (gidx > l_i))
            newmin = jnp.min(jnp.where(valid_new, rowd, _BIG),
                             axis=1, keepdims=True)
            gmin = jnp.where(selg, newmin, gmin)

            logit = 2.0 * (1.0 - jnp.abs(qs - s))
            sig = jax.nn.sigmoid(logit)
            gpos = -jnp.log(sig + 1e-8)
            gneg = -jnp.log(1.0 - sig + 1e-8)
            wpos = jnp.where(t < _KN, 1.0, 0.0)
            sum_pos = sum_pos + wpos * gpos
            sum_neg = sum_neg + (1.0 - wpos) * gneg
            sum8 = sum8 + wpos * s
            return (gmin, m, l_i, sum_pos, sum_neg, sum8)

        return qs, init, step

    qb2 = q_ref.shape[0] // 2
    qs_a, init_a, step_a = make_chain(0, qb2)
    qs_b, init_b, step_b = make_chain(qb2, q_ref.shape[0])

    def body(t, carry):
        return (step_a(t, carry[0]), step_b(t, carry[1]))

    (ca, cb) = jax.lax.fori_loop(0, _K, body, (init_a, init_b))
    _, _, _, pos_a, neg_a, sum8_a = ca
    _, _, _, pos_b, neg_b, sum8_b = cb

    sm_a = (qs_a - sum8_a * (1.0 / _KN)) ** 2
    sm_b = (qs_b - sum8_b * (1.0 / _KN)) ** 2

    @pl.when(i == 0)
    def _init():
        pos_ref[...] = jnp.zeros_like(pos_ref)
        neg_ref[...] = jnp.zeros_like(neg_ref)
        sm_ref[...] = jnp.zeros_like(sm_ref)

    pos_ref[...] += jnp.sum(pos_a) + jnp.sum(pos_b)
    neg_ref[...] += jnp.sum(neg_a) + jnp.sum(neg_b)
    sm_ref[...] += jnp.sum(sm_a) + jnp.sum(sm_b)


def kernel(scores, coords, batch_ids):
    n = scores.shape[0]
    g = n // _L
    grid = (n // _QB,)
    acc = jax.ShapeDtypeStruct((1, 128), jnp.float32)
    pos, neg, sm = pl.pallas_call(
        _knn_loss_kernel,
        grid=grid,
        in_specs=[
            pl.BlockSpec((_QB, 3), lambda i: (i, 0)),
            pl.BlockSpec((_QB, 1), lambda i: (i, 0)),
            pl.BlockSpec((3, g, _L), lambda i: (0, 0, 0)),
            pl.BlockSpec((g, _L), lambda i: (0, 0)),
        ],
        out_specs=[pl.BlockSpec((1, 128), lambda i: (0, 0))] * 3,
        out_shape=[acc, acc, acc],
    )(coords, scores.reshape(n, 1), coords.T.reshape(3, g, _L),
      scores.reshape(g, _L))

    denom = jnp.float32(1.0 / (n * _KN))
    loss_pos = pos[0, 0] * denom
    loss_neg = neg[0, 0] * denom
    loss_contrastive = loss_pos + loss_neg
    loss_smoothness = sm[0, 0] * jnp.float32(1.0 / n)
    loss_locality = jnp.asarray(0.0, dtype=jnp.float32)
    total = (1.0 * loss_locality + 0.5 * loss_contrastive
             + 0.2 * loss_smoothness)
    return (total, loss_locality, loss_contrastive, loss_smoothness)


# speedup vs baseline: 1.1326x; 1.1326x over previous
import jax
import jax.numpy as jnp
from jax.experimental import pallas as pl

_QB = 128
_L = 128
_K = 16
_KN = 8
_BIG = 3.0e38
_IIBIG = 2**30


def _knn_loss_kernel(q_ref, qs_ref, c_ref, s_ref, pos_ref, neg_ref, sm_ref):
    i = pl.program_id(0)
    g = c_ref.shape[1]
    scores_mat = s_ref[...]

    cx = c_ref[0][None]
    cy = c_ref[1][None]
    cz = c_ref[2][None]
    cn = cx * cx + cy * cy + cz * cz

    def make_chain(lo, hi):
        qb = hi - lo
        qx = q_ref[lo:hi, 0:1][:, :, None] * -2.0
        qy = q_ref[lo:hi, 1:2][:, :, None] * -2.0
        qz = q_ref[lo:hi, 2:3][:, :, None] * -2.0
        d2 = cn + (qx * cx + qy * cy + qz * cz)
        gmin0 = jnp.min(d2, axis=2)
        iota_g = jax.lax.broadcasted_iota(jnp.int32, (qb, g), 1)
        iota_l = jax.lax.broadcasted_iota(jnp.int32, (qb, _L), 1)
        qs = qs_ref[lo:hi, 0:1]
        z = jnp.zeros_like(qs)
        init = (gmin0, jnp.full_like(qs, -_BIG),
                jnp.full_like(qs, -1, dtype=jnp.int32), z, z, z)

        def step(t, carry):
            gmin, kd, ki, sum_pos, sum_neg, sum8 = carry
            m = jnp.min(gmin, axis=1, keepdims=True)
            gsel_i = jnp.min(jnp.where(gmin == m, iota_g, _IIBIG),
                             axis=1, keepdims=True)
            selg = iota_g == gsel_i
            penal = (1.0 - selg.astype(jnp.float32)) * _BIG
            rowd = jnp.min(d2 + penal[:, :, None], axis=1)
            gidx = gsel_i * _L + iota_l
            valid = (rowd > kd) | ((rowd == kd) & (gidx > ki))
            l_i = jnp.min(jnp.where(valid & (rowd == m), gidx, _IIBIG),
                          axis=1, keepdims=True)
            srow = jnp.dot(selg.astype(jnp.float32), scores_mat,
                           preferred_element_type=jnp.float32)
            s = jnp.sum(jnp.where(gidx == l_i, srow, 0.0),
                        axis=1, keepdims=True)
            valid_new = (rowd > m) | ((rowd == m) & (gidx > l_i))
            newmin = jnp.min(jnp.where(valid_new, rowd, _BIG),
                             axis=1, keepdims=True)
            gmin = jnp.where(selg, newmin, gmin)

            logit = 2.0 * (1.0 - jnp.abs(qs - s))
            sig = jax.nn.sigmoid(logit)
            gpos = -jnp.log(sig + 1e-8)
            gneg = -jnp.log(1.0 - sig + 1e-8)
            wpos = jnp.where(t < _KN, 1.0, 0.0)
            sum_pos = sum_pos + wpos * gpos
            sum_neg = sum_neg + (1.0 - wpos) * gneg
            sum8 = sum8 + wpos * s
            return (gmin, m, l_i, sum_pos, sum_neg, sum8)

        return qs, init, step

    qs_a, init_a, step_a = make_chain(0, q_ref.shape[0])

    ca = jax.lax.fori_loop(0, _K, step_a, init_a)
    _, _, _, pos_a, neg_a, sum8_a = ca

    sm_a = (qs_a - sum8_a * (1.0 / _KN)) ** 2

    @pl.when(i == 0)
    def _init():
        pos_ref[...] = jnp.zeros_like(pos_ref)
        neg_ref[...] = jnp.zeros_like(neg_ref)
        sm_ref[...] = jnp.zeros_like(sm_ref)

    pos_ref[...] += jnp.sum(pos_a)
    neg_ref[...] += jnp.sum(neg_a)
    sm_ref[...] += jnp.sum(sm_a)


def kernel(scores, coords, batch_ids):
    n = scores.shape[0]
    g = n // _L
    grid = (n // _QB,)
    acc = jax.ShapeDtypeStruct((1, 128), jnp.float32)
    pos, neg, sm = pl.pallas_call(
        _knn_loss_kernel,
        grid=grid,
        in_specs=[
            pl.BlockSpec((_QB, 3), lambda i: (i, 0)),
            pl.BlockSpec((_QB, 1), lambda i: (i, 0)),
            pl.BlockSpec((3, g, _L), lambda i: (0, 0, 0)),
            pl.BlockSpec((g, _L), lambda i: (0, 0)),
        ],
        out_specs=[pl.BlockSpec((1, 128), lambda i: (0, 0))] * 3,
        out_shape=[acc, acc, acc],
    )(coords, scores.reshape(n, 1), coords.T.reshape(3, g, _L),
      scores.reshape(g, _L))

    denom = jnp.float32(1.0 / (n * _KN))
    loss_pos = pos[0, 0] * denom
    loss_neg = neg[0, 0] * denom
    loss_contrastive = loss_pos + loss_neg
    loss_smoothness = sm[0, 0] * jnp.float32(1.0 / n)
    loss_locality = jnp.asarray(0.0, dtype=jnp.float32)
    total = (1.0 * loss_locality + 0.5 * loss_contrastive
             + 0.2 * loss_smoothness)
    return (total, loss_locality, loss_contrastive, loss_smoothness)
